# baseline (speedup 1.0000x reference)
import functools

import jax
import jax.numpy as jnp
from jax import lax
from jax.experimental import pallas as pl
from jax.experimental.pallas import tpu as pltpu
from jax.experimental.pallas import tpu_sc as plsc

NC = 2
NS = 16
NW = NC * NS

NJ = 200
NT = 4096
B = NJ * NT
D = 32
C = 512
TQ = NT // C
NCHUNK = B // C
NCHW = NCHUNK // NW

_MESH = plsc.VectorSubcoreMesh(
    core_axis_name="c", subcore_axis_name="s", num_cores=NC, num_subcores=NS
)

_SCRATCH = (
    [pltpu.VMEM((C,), jnp.int32) for _ in range(2)]
    + [pltpu.VMEM((C, D), jnp.float32) for _ in range(2)]
    + [pltpu.VMEM((D, C), jnp.float32) for _ in range(2)]
    + [pltpu.SemaphoreType.DMA for _ in range(6)]
)


@functools.partial(
    pl.kernel,
    out_type=jax.ShapeDtypeStruct((NJ, D, NT), jnp.float32),
    mesh=_MESH,
    scratch_types=_SCRATCH,
    compiler_params=pltpu.CompilerParams(
        use_tc_tiling_on_sc=False, needs_layout_passes=False
    ),
)
def _embed_gather(idx_hbm, table_hbm, out_hbm, *scratch):
    idxb = scratch[0:2]
    rowsb = scratch[2:4]
    tbufb = scratch[4:6]
    isems = scratch[6:8]
    gsems = scratch[8:10]
    wsems = scratch[10:12]

    wid = lax.axis_index("s") * NC + lax.axis_index("c")
    base = wid * NCHW

    lane = lax.iota(jnp.int32, 16)

    def idx_copy(c, b):
        src = idx_hbm.at[pl.ds((base + c) * C, C)]
        return pltpu.make_async_copy(src, idxb[b], isems[b])

    def gather(c, b):
        return pltpu.make_async_copy(table_hbm.at[idxb[b]], rowsb[b], gsems[b])

    def writeout(c, b):
        gc = base + c
        j = gc // TQ
        t0 = (gc % TQ) * C
        dst = out_hbm.at[j, :, pl.ds(t0, C)]
        return pltpu.make_async_copy(tbufb[b], dst, wsems[b])

    def transpose(b):
        rows = rowsb[b]
        tbuf = tbufb[b]

        def t16_body(t16, _):
            row_vec = lane + t16 * 16
            for d in range(D):
                col_vec = jnp.full((16,), d, jnp.int32)
                g = plsc.load_gather(rows, [row_vec, col_vec])
                tbuf[d, pl.ds(t16 * 16, 16)] = g
            return 0

        lax.fori_loop(0, C // 16, t16_body, 0)

    def body(c, b, first, last):
        idx_copy(c, b).wait()
        gather(c, b).start()
        gather(c - 1, 1 - b).wait()
        if not last:
            idx_copy(c + 1, 1 - b).start()
        if not first:
            writeout(c - 3, 1 - b).wait()
        transpose(1 - b)
        writeout(c - 1, 1 - b).start()

    idx_copy(0, 0).start()
    idx_copy(1, 1).start()
    idx_copy(0, 0).wait()
    gather(0, 0).start()
    body(1, 1, True, False)
    body(2, 0, True, False)

    def main(i, _):
        c = 3 + 2 * i
        body(c, 1, False, False)
        body(c + 1, 0, False, False)
        return 0

    lax.fori_loop(0, (NCHW - 4) // 2, main, 0)

    body(NCHW - 1, 1, False, True)
    gather(NCHW - 1, 1).wait()
    writeout(NCHW - 3, 1).wait()
    transpose(1)
    writeout(NCHW - 1, 1).start()
    writeout(NCHW - 2, 0).wait()
    writeout(NCHW - 1, 1).wait()


def kernel(token_ids, weight):
    flat_p = jnp.transpose(token_ids).reshape(-1).astype(jnp.int32)
    out3 = _embed_gather(flat_p, weight)
    return jnp.transpose(out3, (2, 0, 1))


# speedup vs baseline: 1.4889x; 1.4889x over previous
import functools

import jax
import jax.numpy as jnp
from jax import lax
from jax.experimental import pallas as pl
from jax.experimental.pallas import tpu as pltpu
from jax.experimental.pallas import tpu_sc as plsc

NC = 2
NS = 16
NW = NC * NS

NJ = 200
NT = 4096
B = NJ * NT
D = 32
C = 512
NCHUNK = B // C
NCHW = NCHUNK // NW
TR = 25
TCOL = 32
PADL = 129

_MESH = plsc.VectorSubcoreMesh(
    core_axis_name="c", subcore_axis_name="s", num_cores=NC, num_subcores=NS
)

_SCRATCH = (
    [pltpu.VMEM((C,), jnp.int32) for _ in range(2)]
    + [pltpu.VMEM((C, D), jnp.float32) for _ in range(2)]
    + [pltpu.VMEM((4, D, PADL), jnp.float32) for _ in range(2)]
    + [pltpu.SemaphoreType.DMA for _ in range(6)]
)


@functools.partial(
    pl.kernel,
    out_type=jax.ShapeDtypeStruct((NJ, D, NT), jnp.float32),
    mesh=_MESH,
    scratch_types=_SCRATCH,
    compiler_params=pltpu.CompilerParams(
        use_tc_tiling_on_sc=False, needs_layout_passes=False
    ),
)
def _embed_gather(idx_hbm, table_hbm, out_hbm, *scratch):
    idxb = scratch[0:2]
    rowsb = scratch[2:4]
    tbufb = scratch[4:6]
    isems = scratch[6:8]
    gsems = scratch[8:10]
    wsems = scratch[10:12]

    wid = lax.axis_index("s") * NC + lax.axis_index("c")
    base = wid * NCHW

    iota = lax.iota(jnp.int32, 16)
    d_lo = iota
    d_hi = iota + 16

    def idx_copy(c, b):
        src = idx_hbm.at[pl.ds((base + c) * C, C)]
        return pltpu.make_async_copy(src, idxb[b], isems[b])

    def gather(c, b):
        return pltpu.make_async_copy(table_hbm.at[idxb[b]], rowsb[b], gsems[b])

    def writeout(c, b):
        gc = base + c
        slab = gc // 2
        j0 = (slab // TCOL) * 8 + (gc % 2) * 4
        t0 = (slab % TCOL) * 128
        dst = out_hbm.at[pl.ds(j0, 4), :, pl.ds(t0, 128)]
        return pltpu.make_async_copy(tbufb[b].at[:, :, pl.ds(0, 128)], dst,
                                     wsems[b])

    def transpose(b):
        rows = rowsb[b]
        tbuf = tbufb[b]

        def l16_body(l16, _):
            for r in range(4):
                jr_vec = jnp.full((16,), r, jnp.int32)
                for ll in range(16):
                    lv = l16 * 16 + ll
                    l_vec = jnp.full((16,), lv, jnp.int32)
                    s = r * 128 + lv
                    v0 = rows[s, pl.ds(0, 16)]
                    plsc.store_scatter(tbuf, [jr_vec, d_lo, l_vec], v0)
                    v1 = rows[s, pl.ds(16, 16)]
                    plsc.store_scatter(tbuf, [jr_vec, d_hi, l_vec], v1)
            return 0

        lax.fori_loop(0, 8, l16_body, 0)

    def body(c, b, first, last):
        idx_copy(c, b).wait()
        gather(c, b).start()
        gather(c - 1, 1 - b).wait()
        if not last:
            idx_copy(c + 1, 1 - b).start()
        if not first:
            writeout(c - 3, 1 - b).wait()
        transpose(1 - b)
        writeout(c - 1, 1 - b).start()

    idx_copy(0, 0).start()
    idx_copy(1, 1).start()
    idx_copy(0, 0).wait()
    gather(0, 0).start()
    body(1, 1, True, False)
    body(2, 0, True, False)

    def main(i, _):
        c = 3 + 2 * i
        body(c, 1, False, False)
        body(c + 1, 0, False, False)
        return 0

    lax.fori_loop(0, (NCHW - 4) // 2, main, 0)

    body(NCHW - 1, 1, False, True)
    gather(NCHW - 1, 1).wait()
    writeout(NCHW - 3, 1).wait()
    transpose(1)
    writeout(NCHW - 1, 1).start()
    writeout(NCHW - 2, 0).wait()
    writeout(NCHW - 1, 1).wait()


def kernel(token_ids, weight):
    a = token_ids.reshape(TCOL, 128, TR, 8)
    flat_p = a.transpose(2, 0, 3, 1).reshape(-1).astype(jnp.int32)
    out3 = _embed_gather(flat_p, weight)
    return jnp.transpose(out3, (2, 0, 1))


# speedup vs baseline: 1.5326x; 1.0294x over previous
import functools

import jax
import jax.numpy as jnp
from jax import lax
from jax.experimental import pallas as pl
from jax.experimental.pallas import tpu as pltpu
from jax.experimental.pallas import tpu_sc as plsc

NC = 2
NS = 16
NW = NC * NS

NJ = 200
NT = 4096
B = NJ * NT
D = 32
C = 512
TQ = NT // C
NCHUNK = B // C
NCHW = NCHUNK // NW
PADC = C + 1

_MESH = plsc.VectorSubcoreMesh(
    core_axis_name="c", subcore_axis_name="s", num_cores=NC, num_subcores=NS
)

_SCRATCH = (
    [pltpu.VMEM((C,), jnp.int32) for _ in range(2)]
    + [pltpu.VMEM((C, D), jnp.float32) for _ in range(2)]
    + [pltpu.VMEM((D, PADC), jnp.float32) for _ in range(2)]
    + [pltpu.SemaphoreType.DMA for _ in range(6)]
)


@functools.partial(
    pl.kernel,
    out_type=jax.ShapeDtypeStruct((NJ, D, NT), jnp.float32),
    mesh=_MESH,
    scratch_types=_SCRATCH,
    compiler_params=pltpu.CompilerParams(
        use_tc_tiling_on_sc=False, needs_layout_passes=False
    ),
)
def _embed_gather(idx_hbm, table_hbm, out_hbm, *scratch):
    idxb = scratch[0:2]
    rowsb = scratch[2:4]
    tbufb = scratch[4:6]
    isems = scratch[6:8]
    gsems = scratch[8:10]
    wsems = scratch[10:12]

    wid = lax.axis_index("s") * NC + lax.axis_index("c")
    base = wid * NCHW

    iota = lax.iota(jnp.int32, 16)
    d_lo = iota
    d_hi = iota + 16

    def idx_copy(c, b):
        gc = base + c
        src = idx_hbm.at[gc // TQ, pl.ds((gc % TQ) * C, C)]
        return pltpu.make_async_copy(src, idxb[b], isems[b])

    def gather(c, b):
        return pltpu.make_async_copy(table_hbm.at[idxb[b]], rowsb[b], gsems[b])

    def writeout(c, b):
        gc = base + c
        dst = out_hbm.at[gc // TQ, :, pl.ds((gc % TQ) * C, C)]
        return pltpu.make_async_copy(tbufb[b].at[:, pl.ds(0, C)], dst,
                                     wsems[b])

    def transpose(b):
        rows = rowsb[b]
        tbuf = tbufb[b]

        def l16_body(l16, _):
            for ll in range(16):
                lv = l16 * 16 + ll
                l_vec = jnp.full((16,), lv, jnp.int32)
                v0 = rows[lv, pl.ds(0, 16)]
                plsc.store_scatter(tbuf, [d_lo, l_vec], v0)
                v1 = rows[lv, pl.ds(16, 16)]
                plsc.store_scatter(tbuf, [d_hi, l_vec], v1)
            return 0

        lax.fori_loop(0, C // 16, l16_body, 0)

    def body(c, b, first, last):
        idx_copy(c, b).wait()
        gather(c, b).start()
        gather(c - 1, 1 - b).wait()
        if not last:
            idx_copy(c + 1, 1 - b).start()
        if not first:
            writeout(c - 3, 1 - b).wait()
        transpose(1 - b)
        writeout(c - 1, 1 - b).start()

    idx_copy(0, 0).start()
    idx_copy(1, 1).start()
    idx_copy(0, 0).wait()
    gather(0, 0).start()
    body(1, 1, True, False)
    body(2, 0, True, False)

    def main(i, _):
        c = 3 + 2 * i
        body(c, 1, False, False)
        body(c + 1, 0, False, False)
        return 0

    lax.fori_loop(0, (NCHW - 4) // 2, main, 0)

    body(NCHW - 1, 1, False, True)
    gather(NCHW - 1, 1).wait()
    writeout(NCHW - 3, 1).wait()
    transpose(1)
    writeout(NCHW - 1, 1).start()
    writeout(NCHW - 2, 0).wait()
    writeout(NCHW - 1, 1).wait()


def kernel(token_ids, weight):
    ids_p = jnp.transpose(token_ids).astype(jnp.int32)
    out3 = _embed_gather(ids_p, weight)
    return jnp.transpose(out3, (2, 0, 1))
